# baseline (speedup 1.0000x reference)
import functools

import numpy as np
import jax
import jax.numpy as jnp
from jax import lax
from jax.experimental import pallas as pl
from jax.experimental.pallas import tpu as pltpu
from jax.experimental.pallas import tpu_sc as plsc

_BINS = 64
_SIGMA = 100.0
_DELTA = 1.0 / _BINS
_NF = 16384
_NFP = 16512
_PLANE = 384 * 384
_NPLANES = 6
_NPIX = _PLANE * _NPLANES
_NTILES = 32
_CHUNK = _NPIX // 16
_VECS = _CHUNK // 16
_HSIZE = 2 * _NFP


def _weights() -> np.ndarray:
    q = np.arange(_NF + 1, dtype=np.float64) / _NF
    edges = _DELTA * np.arange(_BINS, dtype=np.float64)
    a = _SIGMA * (q[:, None] - edges[None, :])
    w = 1.0 / (1.0 + np.exp(-a)) - 1.0 / (1.0 + np.exp(-(a - _SIGMA * _DELTA)))
    out = np.zeros((_NFP, _BINS), dtype=np.float32)
    out[: _NF + 1] = w.astype(np.float32)
    return out


def _fold_matrix() -> np.ndarray:
    a = np.zeros((_NPLANES, 2 * _NTILES), dtype=np.float32)
    for wid in range(_NTILES):
        sign = 1.0 if wid < 16 else -1.0
        tid = wid % 16
        s = tid * _CHUNK
        p0 = s // _PLANE
        a[p0, 2 * wid] = sign
        if (s + _CHUNK - 1) // _PLANE > p0:
            a[p0 + 1, 2 * wid + 1] = sign
    return a


_W_NP = _weights()
_A_NP = _fold_matrix()


def _sc_hist_body(out_arr, tgt_arr, part, pix, hist, sem):
    c = lax.axis_index("c")
    s = lax.axis_index("s")
    wid = s * 2 + c
    aid = wid // 16
    tid = wid % 16
    base = tid * _CHUNK
    p0 = base // _PLANE
    bvec = (jnp.minimum((p0 + 1) * _PLANE, base + _CHUNK) - base) // 16

    @pl.when(aid == 0)
    def _():
        pltpu.async_copy(out_arr.at[pl.ds(base, _CHUNK)], pix, sem)

    @pl.when(aid == 1)
    def _():
        pltpu.async_copy(tgt_arr.at[pl.ds(base, _CHUNK)], pix, sem)

    zero = jnp.zeros((16,), jnp.float32)

    def zbody(k, carry):
        for m in range(8):
            hist[pl.ds((k * 8 + m) * 16, 16)] = zero
        return carry

    lax.fori_loop(0, _HSIZE // 128, zbody, 0)

    pltpu.make_async_copy(out_arr.at[pl.ds(base, _CHUNK)], pix, sem).wait()

    ones = jnp.full((16,), 1.0, jnp.float32)

    def _scatter_loop(nvecs, vbase, off):
        def body(j, carry):
            for m in range(4):
                v = pix[pl.ds((vbase + j * 4 + m) * 16, 16)]
                u = v * float(_NF) + 0.5
                i = u.astype(jnp.int32)
                plsc.addupdate_scatter(hist, [i + off if off else i], ones)
            return carry

        lax.fori_loop(0, nvecs // 4, body, 0)

    _scatter_loop(bvec, 0, 0)
    _scatter_loop(_VECS - bvec, bvec, _NFP)

    pltpu.sync_copy(hist, part.at[wid])


@functools.cache
def _sc_hist():
    return pl.kernel(
        _sc_hist_body,
        out_type=jax.ShapeDtypeStruct((_NTILES, _HSIZE), jnp.float32),
        mesh=plsc.VectorSubcoreMesh(core_axis_name="c", subcore_axis_name="s"),
        scratch_types=[
            pltpu.VMEM((_CHUNK,), jnp.float32),
            pltpu.VMEM((_HSIZE,), jnp.float32),
            pltpu.SemaphoreType.DMA,
        ],
        compiler_params=pltpu.CompilerParams(needs_layout_passes=False),
    )


def _tc_loss_body(part_ref, w_ref, a_ref, out_ref):
    d = jnp.dot(a_ref[...], part_ref[...], preferred_element_type=jnp.float32)
    h = jnp.dot(d, w_ref[...], preferred_element_type=jnp.float32)
    loss = jnp.sum(jnp.abs(h)) * (1.0 / (_NPLANES * _BINS * _PLANE))
    out_ref[...] = jnp.reshape(loss, (1, 1))


def kernel(output, target):
    part = _sc_hist()(output.reshape(-1), target.reshape(-1))
    part2 = part.reshape(2 * _NTILES, _NFP)
    loss = pl.pallas_call(
        _tc_loss_body,
        out_shape=jax.ShapeDtypeStruct((1, 1), jnp.float32),
    )(part2, jnp.asarray(_W_NP), jnp.asarray(_A_NP))
    return loss[0, 0]


# speedup vs baseline: 1.7229x; 1.7229x over previous
import functools

import numpy as np
import jax
import jax.numpy as jnp
from jax import lax
from jax.experimental import pallas as pl
from jax.experimental.pallas import tpu as pltpu
from jax.experimental.pallas import tpu_sc as plsc

_BINS = 64
_SIGMA = 100.0
_DELTA = 1.0 / _BINS
_NF = 16384
_NFP = 16512
_PLANE = 384 * 384
_NPLANES = 6
_NPIX = _PLANE * _NPLANES
_NTILES = 32
_CHUNK = _NPIX // 16
_VECS = _CHUNK // 16
_HSIZE = 2 * _NFP


def _weights() -> np.ndarray:
    q = np.arange(_NF + 1, dtype=np.float64) / _NF
    edges = _DELTA * np.arange(_BINS, dtype=np.float64)
    a = _SIGMA * (q[:, None] - edges[None, :])
    w = 1.0 / (1.0 + np.exp(-a)) - 1.0 / (1.0 + np.exp(-(a - _SIGMA * _DELTA)))
    out = np.zeros((_NFP, _BINS), dtype=np.float32)
    out[: _NF + 1] = w.astype(np.float32)
    return out


def _fold_matrix() -> np.ndarray:
    a = np.zeros((_NPLANES, 2 * _NTILES), dtype=np.float32)
    for wid in range(_NTILES):
        sign = 1.0 if wid < 16 else -1.0
        tid = wid % 16
        s = tid * _CHUNK
        p0 = s // _PLANE
        a[p0, 2 * wid] = sign
        if (s + _CHUNK - 1) // _PLANE > p0:
            a[p0 + 1, 2 * wid + 1] = sign
    return a


_W_NP = _weights()
_A_NP = _fold_matrix()


def _sc_hist_body(out_arr, tgt_arr, part, pix, hist, sem):
    c = lax.axis_index("c")
    s = lax.axis_index("s")
    wid = s * 2 + c
    aid = wid // 16
    tid = wid % 16
    base = tid * _CHUNK
    p0 = base // _PLANE
    bvec = (jnp.minimum((p0 + 1) * _PLANE, base + _CHUNK) - base) // 16

    @pl.when(aid == 0)
    def _():
        pltpu.async_copy(out_arr.at[pl.ds(base, _CHUNK)], pix, sem)

    @pl.when(aid == 1)
    def _():
        pltpu.async_copy(tgt_arr.at[pl.ds(base, _CHUNK)], pix, sem)

    zero = jnp.zeros((16,), jnp.float32)

    @plsc.parallel_loop(0, _HSIZE // 16, unroll=8)
    def _(k):
        hist[pl.ds(k * 16, 16)] = zero

    pltpu.make_async_copy(out_arr.at[pl.ds(base, _CHUNK)], pix, sem).wait()

    ones = jnp.full((16,), 1.0, jnp.float32)

    def _scatter_loop(nvecs, vbase, off):
        @plsc.parallel_loop(0, nvecs, unroll=8)
        def _(j):
            v = pix[pl.ds((vbase + j) * 16, 16)]
            u = v * float(_NF) + 0.5
            i = u.astype(jnp.int32)
            plsc.addupdate_scatter(hist, [i + off if off else i], ones)

    _scatter_loop(bvec, 0, 0)
    _scatter_loop(_VECS - bvec, bvec, _NFP)

    pltpu.sync_copy(hist, part.at[wid])


@functools.cache
def _sc_hist():
    return pl.kernel(
        _sc_hist_body,
        out_type=jax.ShapeDtypeStruct((_NTILES, _HSIZE), jnp.float32),
        mesh=plsc.VectorSubcoreMesh(core_axis_name="c", subcore_axis_name="s"),
        scratch_types=[
            pltpu.VMEM((_CHUNK,), jnp.float32),
            pltpu.VMEM((_HSIZE,), jnp.float32),
            pltpu.SemaphoreType.DMA,
        ],
        compiler_params=pltpu.CompilerParams(needs_layout_passes=False),
    )


def _tc_loss_body(part_ref, w_ref, a_ref, out_ref):
    d = jnp.dot(a_ref[...], part_ref[...], preferred_element_type=jnp.float32)
    h = jnp.dot(d, w_ref[...], preferred_element_type=jnp.float32)
    loss = jnp.sum(jnp.abs(h)) * (1.0 / (_NPLANES * _BINS * _PLANE))
    out_ref[...] = jnp.reshape(loss, (1, 1))


def kernel(output, target):
    part = _sc_hist()(output.reshape(-1), target.reshape(-1))
    part2 = part.reshape(2 * _NTILES, _NFP)
    loss = pl.pallas_call(
        _tc_loss_body,
        out_shape=jax.ShapeDtypeStruct((1, 1), jnp.float32),
    )(part2, jnp.asarray(_W_NP), jnp.asarray(_A_NP))
    return loss[0, 0]


# speedup vs baseline: 2.0818x; 1.2083x over previous
import functools

import numpy as np
import jax
import jax.numpy as jnp
from jax import lax
from jax.experimental import pallas as pl
from jax.experimental.pallas import tpu as pltpu
from jax.experimental.pallas import tpu_sc as plsc

_BINS = 64
_SIGMA = 100.0
_DELTA = 1.0 / _BINS
_NF = 8192
_NFP = 8320
_W = 384
_ROWS_PER_PLANE = 384
_NROWS = 6 * _ROWS_PER_PLANE
_PLANE = _ROWS_PER_PLANE * _W
_NPLANES = 6
_NTILES = 32
_TROWS = _NROWS // 16
_RVECS = _W // 16
_HSIZE = 2 * _NFP


def _weights() -> np.ndarray:
    q = np.arange(_NF + 1, dtype=np.float64) / _NF
    edges = _DELTA * np.arange(_BINS, dtype=np.float64)
    a = _SIGMA * (q[:, None] - edges[None, :])
    w = 1.0 / (1.0 + np.exp(-a)) - 1.0 / (1.0 + np.exp(-(a - _SIGMA * _DELTA)))
    out = np.zeros((_NFP, _BINS), dtype=np.float32)
    out[: _NF + 1] = w.astype(np.float32)
    return out


def _fold_matrix() -> np.ndarray:
    a = np.zeros((_NPLANES, 2 * _NTILES), dtype=np.float32)
    for wid in range(_NTILES):
        sign = 1.0 if wid < 16 else -1.0
        tid = wid % 16
        r = tid * _TROWS
        p0 = r // _ROWS_PER_PLANE
        a[p0, 2 * wid] = sign
        if (r + _TROWS - 1) // _ROWS_PER_PLANE > p0:
            a[p0 + 1, 2 * wid + 1] = sign
    return a


_W_NP = _weights()
_A_NP = _fold_matrix()


def _sc_hist_body(out_arr, tgt_arr, part, pix, hist, sem):
    c = lax.axis_index("c")
    s = lax.axis_index("s")
    wid = s * 2 + c
    aid = wid // 16
    tid = wid % 16
    row0 = tid * _TROWS
    p0 = row0 // _ROWS_PER_PLANE
    brow = jnp.minimum((p0 + 1) * _ROWS_PER_PLANE, row0 + _TROWS) - row0

    @pl.when(aid == 0)
    def _():
        pltpu.async_copy(out_arr.at[pl.ds(row0, _TROWS), :], pix, sem)

    @pl.when(aid == 1)
    def _():
        pltpu.async_copy(tgt_arr.at[pl.ds(row0, _TROWS), :], pix, sem)

    zero = jnp.zeros((16,), jnp.float32)

    @plsc.parallel_loop(0, _HSIZE // 16, unroll=8)
    def _(k):
        hist[pl.ds(k * 16, 16)] = zero

    pltpu.make_async_copy(out_arr.at[pl.ds(row0, _TROWS), :], pix, sem).wait()

    ones = jnp.full((16,), 1.0, jnp.float32)

    def _scatter_rows(nrows, rbase, off):
        @plsc.parallel_loop(0, nrows, unroll=2)
        def _(r):
            row = rbase + r
            for k in range(_RVECS):
                v = pix[row, pl.ds(k * 16, 16)]
                u = v * float(_NF) + 0.5
                i = u.astype(jnp.int32)
                plsc.addupdate_scatter(hist, [i + off if off else i], ones)

    _scatter_rows(brow, 0, 0)
    _scatter_rows(_TROWS - brow, brow, _NFP)

    sync = pltpu.sync_copy
    sync(hist.at[pl.ds(0, _NFP)], part.at[2 * wid])
    sync(hist.at[pl.ds(_NFP, _NFP)], part.at[2 * wid + 1])


@functools.cache
def _sc_hist():
    return pl.kernel(
        _sc_hist_body,
        out_type=jax.ShapeDtypeStruct((2 * _NTILES, _NFP), jnp.float32),
        mesh=plsc.VectorSubcoreMesh(core_axis_name="c", subcore_axis_name="s"),
        scratch_types=[
            pltpu.VMEM((_TROWS, _W), jnp.float32),
            pltpu.VMEM((_HSIZE,), jnp.float32),
            pltpu.SemaphoreType.DMA,
        ],
        compiler_params=pltpu.CompilerParams(needs_layout_passes=False),
    )


def _tc_loss_body(part_ref, w_ref, a_ref, out_ref):
    d = jnp.dot(a_ref[...], part_ref[...], preferred_element_type=jnp.float32)
    h = jnp.dot(d, w_ref[...], preferred_element_type=jnp.float32)
    loss = jnp.sum(jnp.abs(h)) * (1.0 / (_NPLANES * _BINS * _PLANE))
    out_ref[...] = jnp.reshape(loss, (1, 1))


def kernel(output, target):
    o2 = output.reshape(_NROWS, _W)
    t2 = target.reshape(_NROWS, _W)
    part = _sc_hist()(o2, t2)
    loss = pl.pallas_call(
        _tc_loss_body,
        out_shape=jax.ShapeDtypeStruct((1, 1), jnp.float32),
    )(part, jnp.asarray(_W_NP), jnp.asarray(_A_NP))
    return loss[0, 0]
